# baseline (speedup 1.0000x reference)
import functools

import jax
import jax.numpy as jnp
from jax import lax
from jax.experimental import pallas as pl
from jax.experimental.pallas import tpu as pltpu
from jax.experimental.pallas import tpu_sc as plsc

_NUM_CORES = 2
_NUM_SUBCORES = 16
_NUM_WORKERS = _NUM_CORES * _NUM_SUBCORES
_RB = 4


def _sload(ref, i):
    return ref[pl.ds(i, 16)][0]


def _sstore(ref, i, val_vec16, lane0_mask):
    plsc.store_scatter(ref, [jnp.full((16,), i, jnp.int32)], val_vec16, mask=lane0_mask)


def _inv_kernel(
    N, V, D, idx_hbm, table_hbm, out_hbm, idx_all, punsort, plist, cursor,
    buf0, buf1, rsem0, rsem1, wsem0, wsem1
):
    bufs = (buf0, buf1)
    rsems = (rsem0, rsem1)
    wsems = (wsem0, wsem1)
    wid = lax.axis_index("s") * _NUM_CORES + lax.axis_index("c")
    vr = V // _NUM_WORKERS
    nch = vr // _RB
    lo = wid * vr

    lane = lax.iota(jnp.int32, 16)
    lane0 = lane == 0
    ones = jnp.ones((16,), jnp.int32)

    pltpu.sync_copy(idx_hbm, idx_all.at[pl.ds(0, N)])

    for b in range(2):
        pltpu.async_copy(table_hbm.at[pl.ds(lo + b * _RB, _RB)], bufs[b], rsems[b])

    @pl.loop(0, N // 16, init_carry=0, unroll=2)
    def count(i, cnt):
        vals = idx_all[pl.ds(i * 16, 16)]
        mask = (vals >= lo) & (vals < lo + vr)
        packed = vals * 16384 + (lane + i * 16)
        cs = plsc.cumsum(mask.astype(jnp.int32))
        plsc.store_scatter(punsort, [cnt + cs - 1], packed, mask=mask)
        return cnt + cs[15]

    @pl.loop(0, 4)
    def _zero(i):
        cursor[pl.ds(i * 16, 16)] = jnp.zeros((16,), jnp.int32)

    @pl.loop(0, count)
    def _hist(j):
        w = _sload(punsort, j)
        b = (w // 16384 - lo) // _RB
        sv = cursor[pl.ds(b + 1, 16)]
        _sstore(cursor, b + 1, sv + 1, lane0)

    @pl.loop(0, nch)
    def _scan(b):
        sv = cursor[pl.ds(b, 16)]
        nxt = cursor[pl.ds(b + 1, 16)]
        _sstore(cursor, b + 1, nxt + sv[0], lane0)

    @pl.loop(0, count)
    def _place(j):
        wv = punsort[pl.ds(j, 16)]
        w = wv[0]
        b = (w // 16384 - lo) // _RB
        sv = cursor[pl.ds(b, 16)]
        _sstore(plist, sv[0], wv, lane0)
        _sstore(cursor, b, sv + 1, lane0)

    @pl.loop(0, nch, step=2, init_carry=0)
    def _chunks(c0, start):
        for b in range(2):
            c = c0 + b
            buf = bufs[b]
            pltpu.make_async_copy(
                table_hbm.at[pl.ds(lo + c * _RB, _RB)], buf, rsems[b]
            ).wait()
            endc = _sload(cursor, c)

            @pl.loop(start, endc)
            def _scatter(j):
                w = _sload(plist, j)
                p = w % 16384
                r = w // 16384 - (lo + c * _RB)
                pltpu.async_copy(buf.at[pl.ds(r, 1)], out_hbm.at[pl.ds(p, 1)], wsems[b])

            @pl.loop(start, endc)
            def _drain(j):
                pltpu.make_async_copy(
                    buf.at[pl.ds(0, 1)], out_hbm.at[pl.ds(0, 1)], wsems[b]
                ).wait()

            @pl.when(c + 2 < nch)
            def _next_read():
                pltpu.async_copy(
                    table_hbm.at[pl.ds(lo + (c + 2) * _RB, _RB)], buf, rsems[b]
                )

            start = endc
        return start


def kernel(token_indices, table):
    B, T = token_indices.shape
    V, D = table.shape
    N = B * T

    mesh = plsc.VectorSubcoreMesh(
        core_axis_name="c",
        subcore_axis_name="s",
        num_cores=_NUM_CORES,
        num_subcores=_NUM_SUBCORES,
    )

    run = pl.kernel(
        functools.partial(_inv_kernel, N, V, D),
        out_type=jax.ShapeDtypeStruct((N, D), jnp.float32),
        mesh=mesh,
        compiler_params=pltpu.CompilerParams(needs_layout_passes=False),
        scratch_types=[
            pltpu.VMEM((N + 16,), jnp.int32),
            pltpu.VMEM((N + 32,), jnp.int32),
            pltpu.VMEM((N + 32,), jnp.int32),
            pltpu.VMEM((64,), jnp.int32),
            pltpu.VMEM((_RB, D), jnp.float32),
            pltpu.VMEM((_RB, D), jnp.float32),
            pltpu.SemaphoreType.DMA,
            pltpu.SemaphoreType.DMA,
            pltpu.SemaphoreType.DMA,
            pltpu.SemaphoreType.DMA,
        ],
    )
    out = run(token_indices.reshape(N), table)
    return out.reshape(B, T, D)


# speedup vs baseline: 1.0368x; 1.0368x over previous
import functools

import jax
import jax.numpy as jnp
from jax import lax
from jax.experimental import pallas as pl
from jax.experimental.pallas import tpu as pltpu
from jax.experimental.pallas import tpu_sc as plsc

_NUM_CORES = 2
_NUM_SUBCORES = 16
_NUM_WORKERS = _NUM_CORES * _NUM_SUBCORES
_RB = 4


def _sload(ref, i):
    return ref[pl.ds(i, 16)][0]


def _sstore(ref, i, val_vec16, lane0_mask):
    plsc.store_scatter(ref, [jnp.full((16,), i, jnp.int32)], val_vec16, mask=lane0_mask)


def _inv_kernel(
    N, V, D, idx_hbm, table_hbm, out_hbm, idx_all, punsort, plist, cursor,
    buf0, buf1, rsem0, rsem1, wsem0, wsem1
):
    bufs = (buf0, buf1)
    rsems = (rsem0, rsem1)
    wsems = (wsem0, wsem1)
    wid = lax.axis_index("s") * _NUM_CORES + lax.axis_index("c")
    vr = V // _NUM_WORKERS
    nch = vr // _RB
    lo = wid * vr

    lane = lax.iota(jnp.int32, 16)
    lane0 = lane == 0
    ones = jnp.ones((16,), jnp.int32)

    pltpu.sync_copy(idx_hbm, idx_all.at[pl.ds(0, N)])

    for b in range(2):
        pltpu.async_copy(table_hbm.at[pl.ds(lo + b * _RB, _RB)], bufs[b], rsems[b])

    @pl.loop(0, N // 16, init_carry=0, unroll=2)
    def count(i, cnt):
        vals = idx_all[pl.ds(i * 16, 16)]
        mask = (vals >= lo) & (vals < lo + vr)
        packed = vals * 16384 + (lane + i * 16)
        cs = plsc.cumsum(mask.astype(jnp.int32))
        plsc.store_scatter(punsort, [cnt + cs - 1], packed, mask=mask)
        return cnt + cs[15]

    @pl.loop(0, 4)
    def _zero(i):
        cursor[pl.ds(i * 16, 16)] = jnp.zeros((16,), jnp.int32)

    @pl.loop(0, (count + 15) // 16)
    def _hist(g):
        in_range = g * 16 + lane < count
        w = punsort[pl.ds(g * 16, 16)]
        bins = (w // 16384 - lo) // _RB + 1
        plsc.addupdate_scatter(cursor, [jnp.where(in_range, bins, 63)], ones, mask=in_range)

    @pl.loop(0, nch)
    def _scan(b):
        sv = cursor[pl.ds(b, 16)]
        nxt = cursor[pl.ds(b + 1, 16)]
        _sstore(cursor, b + 1, nxt + sv[0], lane0)

    @pl.loop(0, count)
    def _place(j):
        wv = punsort[pl.ds(j, 16)]
        w = wv[0]
        b = (w // 16384 - lo) // _RB
        sv = cursor[pl.ds(b, 16)]
        _sstore(plist, sv[0], wv, lane0)
        _sstore(cursor, b, sv + 1, lane0)

    @pl.loop(0, nch, step=2, init_carry=0)
    def _chunks(c0, start):
        for b in range(2):
            c = c0 + b
            buf = bufs[b]
            pltpu.make_async_copy(
                table_hbm.at[pl.ds(lo + c * _RB, _RB)], buf, rsems[b]
            ).wait()
            endc = _sload(cursor, c)

            @pl.loop(start, endc)
            def _scatter(j):
                w = _sload(plist, j)
                p = w % 16384
                r = w // 16384 - (lo + c * _RB)
                pltpu.async_copy(buf.at[pl.ds(r, 1)], out_hbm.at[pl.ds(p, 1)], wsems[b])

            @pl.loop(start, endc)
            def _drain(j):
                pltpu.make_async_copy(
                    buf.at[pl.ds(0, 1)], out_hbm.at[pl.ds(0, 1)], wsems[b]
                ).wait()

            @pl.when(c + 2 < nch)
            def _next_read():
                pltpu.async_copy(
                    table_hbm.at[pl.ds(lo + (c + 2) * _RB, _RB)], buf, rsems[b]
                )

            start = endc
        return start


def kernel(token_indices, table):
    B, T = token_indices.shape
    V, D = table.shape
    N = B * T

    mesh = plsc.VectorSubcoreMesh(
        core_axis_name="c",
        subcore_axis_name="s",
        num_cores=_NUM_CORES,
        num_subcores=_NUM_SUBCORES,
    )

    run = pl.kernel(
        functools.partial(_inv_kernel, N, V, D),
        out_type=jax.ShapeDtypeStruct((N, D), jnp.float32),
        mesh=mesh,
        compiler_params=pltpu.CompilerParams(needs_layout_passes=False),
        scratch_types=[
            pltpu.VMEM((N + 16,), jnp.int32),
            pltpu.VMEM((N + 32,), jnp.int32),
            pltpu.VMEM((N + 32,), jnp.int32),
            pltpu.VMEM((64,), jnp.int32),
            pltpu.VMEM((_RB, D), jnp.float32),
            pltpu.VMEM((_RB, D), jnp.float32),
            pltpu.SemaphoreType.DMA,
            pltpu.SemaphoreType.DMA,
            pltpu.SemaphoreType.DMA,
            pltpu.SemaphoreType.DMA,
        ],
    )
    out = run(token_indices.reshape(N), table)
    return out.reshape(B, T, D)
